# baseline (speedup 1.0000x reference)
import functools

import jax
import jax.numpy as jnp
from jax import lax
from jax.experimental import pallas as pl
from jax.experimental.pallas import tpu as pltpu
from jax.experimental.pallas import tpu_sc as plsc

_F32 = jnp.float32
_CHUNK = 64
_NBUF = 4
_DEG_W = 16
_G = 64


def _row_split(n_rows, n_tiles):
    base = -(-n_rows // n_tiles)
    base = -(-base // 8) * 8
    out = []
    start = 0
    for t in range(n_tiles):
        sz = min(base, n_rows - start)
        if sz <= 0:
            break
        out.append((start, sz))
        start += sz
    return out


def _copy_chunks(src_ref, dst_ref, dst_start, total, buf_rows):
    off = 0
    while off < total:
        sz = min(buf_rows, total - off)
        pltpu.sync_copy(src_ref.at[pl.ds(0, sz)], dst_ref.at[pl.ds(dst_start + off, sz)])
        off += sz


def _degree_call(dst_p, n, interpret=False):
    rows_total = dst_p.shape[0]
    per_core = rows_total // 2
    per_tile = per_core // 16
    acc_rows = n + 8
    out_split = _row_split(n, 16)
    zero_split = _row_split(acc_rows, 16)
    mesh = plsc.VectorSubcoreMesh(
        core_axis_name="c", subcore_axis_name="s", num_cores=2, num_subcores=16)

    @functools.partial(
        pl.kernel,
        out_type=[jax.ShapeDtypeStruct((n, _DEG_W), _F32)] * 2,
        mesh=mesh,
        interpret=interpret,
        scratch_types=[
            pltpu.VMEM((per_tile, _CHUNK), jnp.int32),
            pltpu.VMEM((_CHUNK, _DEG_W), _F32),
            pltpu.VMEM((_CHUNK, _DEG_W), _F32),
            pltpu.VMEM_SHARED((acc_rows, _DEG_W), _F32),
        ],
    )
    def deg_kernel(dst_hbm, d0_hbm, d1_hbm, didx, ones_v, zb, acc):
        cid = lax.axis_index("c")
        tid = lax.axis_index("s")

        pltpu.sync_copy(
            dst_hbm.at[pl.ds(cid * per_core + tid * per_tile, per_tile)], didx)

        def init_row(i, _):
            ones_v[i, :] = jnp.ones((_DEG_W,), _F32)
            zb[i, :] = jnp.zeros((_DEG_W,), _F32)
            return ()

        lax.fori_loop(0, _CHUNK, init_row, ())

        for t, (zs, zn) in enumerate(zero_split):
            @pl.when(tid == t)
            def _(zs=zs, zn=zn):
                _copy_chunks(zb, acc, zs, zn, _CHUNK)

        plsc.subcore_barrier()

        def body(i, _):
            pltpu.sync_copy(ones_v, acc.at[didx.at[i]], add=True)
            return ()

        lax.fori_loop(0, per_tile, body, ())
        plsc.subcore_barrier()

        def copy_out(out_hbm):
            for t, (os, on) in enumerate(out_split):
                @pl.when(tid == t)
                def _(os=os, on=on):
                    pltpu.sync_copy(acc.at[pl.ds(os, on)], out_hbm.at[pl.ds(os, on)])

        @pl.when(cid == 0)
        def _():
            copy_out(d0_hbm)

        @pl.when(cid == 1)
        def _():
            copy_out(d1_hbm)

    return deg_kernel(dst_p)


def _aggregate_call(src_p, dst_p, p0, p1, interpret=False):
    n = p0.shape[0]
    hw = p0.shape[1]
    per_tile = src_p.shape[0] // 16
    acc_rows = n + 8
    out_split = _row_split(n, 16)
    zero_split = _row_split(acc_rows, 16)
    mesh = plsc.VectorSubcoreMesh(
        core_axis_name="c", subcore_axis_name="s", num_cores=2, num_subcores=16)

    quarter = per_tile // 4
    zrows = 16

    @functools.partial(
        pl.kernel,
        out_type=[jax.ShapeDtypeStruct((n, hw), _F32)] * 2,
        mesh=mesh,
        interpret=interpret,
        scratch_types=[
            pltpu.VMEM((quarter, _CHUNK), jnp.int32),
            pltpu.VMEM((quarter, _CHUNK), jnp.int32),
            pltpu.VMEM((_CHUNK, hw), _F32),
            pltpu.VMEM((_CHUNK, hw), _F32),
            pltpu.VMEM((_CHUNK, hw), _F32),
            pltpu.VMEM((_CHUNK, hw), _F32),
            pltpu.VMEM((zrows, hw), _F32),
            pltpu.VMEM_SHARED((acc_rows, hw), _F32),
            pltpu.SemaphoreType.DMA,
            pltpu.SemaphoreType.DMA,
            pltpu.SemaphoreType.DMA,
            pltpu.SemaphoreType.DMA,
        ],
    )
    def agg_kernel(src_hbm, dst_hbm, p0_hbm, p1_hbm, s0_hbm, s1_hbm,
                   sidx, didx, rows0, rows1, rows2, rows3, zb, acc,
                   sem0, sem1, sem2, sem3):
        cid = lax.axis_index("c")
        tid = lax.axis_index("s")

        def zero_row(i, _):
            for j in range(hw // 16):
                zb[i, pl.ds(j * 16, 16)] = jnp.zeros((16,), _F32)
            return ()

        lax.fori_loop(0, zrows, zero_row, ())

        for t, (zs, zn) in enumerate(zero_split):
            @pl.when(tid == t)
            def _(zs=zs, zn=zn):
                _copy_chunks(zb, acc, zs, zn, zrows)

        plsc.subcore_barrier()

        def edge_loop(table_hbm):
            bufs = (rows0, rows1, rows2, rows3)
            sems = (sem0, sem1, sem2, sem3)
            for q in range(4):
                base = tid * per_tile + q * quarter
                pltpu.sync_copy(src_hbm.at[pl.ds(base, quarter)], sidx)
                pltpu.sync_copy(dst_hbm.at[pl.ds(base, quarter)], didx)
                for b in range(_NBUF):
                    pltpu.async_copy(table_hbm.at[sidx.at[b]], bufs[b], sems[b])

                def body(g, _):
                    for b in range(_NBUF):
                        cur = g * _NBUF + b
                        pltpu.make_async_copy(
                            table_hbm.at[pl.ds(0, _CHUNK)],
                            bufs[b], sems[b]).wait()
                        pltpu.sync_copy(bufs[b], acc.at[didx.at[cur]], add=True)
                        nxt = cur + _NBUF

                        @pl.when(nxt < quarter)
                        def _(b=b, nxt=nxt):
                            pltpu.async_copy(
                                table_hbm.at[sidx.at[nxt]], bufs[b], sems[b])
                    return ()

                lax.fori_loop(0, quarter // _NBUF, body, ())

        @pl.when(cid == 0)
        def _():
            edge_loop(p0_hbm)

        @pl.when(cid == 1)
        def _():
            edge_loop(p1_hbm)

        plsc.subcore_barrier()

        def copy_out(out_hbm):
            for t, (os, on) in enumerate(out_split):
                @pl.when(tid == t)
                def _(os=os, on=on):
                    pltpu.sync_copy(acc.at[pl.ds(os, on)], out_hbm.at[pl.ds(os, on)])

        @pl.when(cid == 0)
        def _():
            copy_out(s0_hbm)

        @pl.when(cid == 1)
        def _():
            copy_out(s1_hbm)

    return agg_kernel(src_p, dst_p, p0, p1)


def _stage0_call(x, w1, d0, d1, r, interpret=False):
    n, d = x.shape

    def body(x_ref, w_ref, d0_ref, d1_ref, p0_ref, p1_ref, dv_ref):
        dinv = lax.rsqrt(d0_ref[:, 0:1] + d1_ref[:, 0:1] + 1.0)
        y = jnp.dot(x_ref[...], w_ref[...], preferred_element_type=_F32)
        p = y * dinv
        p0_ref[...] = p[:, : d // 2]
        p1_ref[...] = p[:, d // 2:]
        dv_ref[...] = dinv

    return pl.pallas_call(
        body,
        grid=(n // r,),
        in_specs=[
            pl.BlockSpec((r, d), lambda i: (i, 0)),
            pl.BlockSpec((d, d), lambda i: (0, 0)),
            pl.BlockSpec((r, _DEG_W), lambda i: (i, 0)),
            pl.BlockSpec((r, _DEG_W), lambda i: (i, 0)),
        ],
        out_specs=[
            pl.BlockSpec((r, d // 2), lambda i: (i, 0)),
            pl.BlockSpec((r, d // 2), lambda i: (i, 0)),
            pl.BlockSpec((r, 1), lambda i: (i, 0)),
        ],
        out_shape=[
            jax.ShapeDtypeStruct((n, d // 2), _F32),
            jax.ShapeDtypeStruct((n, d // 2), _F32),
            jax.ShapeDtypeStruct((n, 1), _F32),
        ],
        interpret=interpret,
    )(x, w1, d0, d1)


def _stage_mid_call(s0, s1, p0, p1, dv, b, w, r, interpret=False):
    n, hw = s0.shape
    d = 2 * hw

    def body(s0_ref, s1_ref, p0_ref, p1_ref, dv_ref, b_ref, w_ref, q0_ref, q1_ref):
        t = jnp.concatenate(
            [s0_ref[...] + p0_ref[...], s1_ref[...] + p1_ref[...]], axis=1)
        h = jnp.maximum(dv_ref[...] * t + b_ref[...], 0.0)
        y = jnp.dot(h, w_ref[...], preferred_element_type=_F32)
        q = y * dv_ref[...]
        q0_ref[...] = q[:, :hw]
        q1_ref[...] = q[:, hw:]

    half = pl.BlockSpec((r, hw), lambda i: (i, 0))
    return pl.pallas_call(
        body,
        grid=(n // r,),
        in_specs=[half, half, half, half,
                  pl.BlockSpec((r, 1), lambda i: (i, 0)),
                  pl.BlockSpec((1, d), lambda i: (0, 0)),
                  pl.BlockSpec((d, d), lambda i: (0, 0))],
        out_specs=[half, half],
        out_shape=[jax.ShapeDtypeStruct((n, hw), _F32)] * 2,
        interpret=interpret,
    )(s0, s1, p0, p1, dv, b, w)


def _stage_final_call(s0, s1, p0, p1, dv, b, batch_blocks, r, interpret=False):
    n, hw = s0.shape
    d = 2 * hw

    def body(s0_ref, s1_ref, p0_ref, p1_ref, dv_ref, b_ref, bat_ref, g_ref):
        i = pl.program_id(0)
        t = jnp.concatenate(
            [s0_ref[...] + p0_ref[...], s1_ref[...] + p1_ref[...]], axis=1)
        h = jnp.maximum(dv_ref[...] * t + b_ref[...], 0.0)
        ids = jnp.broadcast_to(bat_ref[0], (_G, r))
        onehot = (lax.broadcasted_iota(jnp.int32, (_G, r), 0) == ids).astype(_F32)
        gp = jnp.dot(onehot, h, preferred_element_type=_F32)

        @pl.when(i == 0)
        def _():
            g_ref[...] = gp

        @pl.when(i > 0)
        def _():
            g_ref[...] += gp

    half = pl.BlockSpec((r, hw), lambda i: (i, 0))
    return pl.pallas_call(
        body,
        grid=(n // r,),
        in_specs=[half, half, half, half,
                  pl.BlockSpec((r, 1), lambda i: (i, 0)),
                  pl.BlockSpec((1, d), lambda i: (0, 0)),
                  pl.BlockSpec((1, 1, r), lambda i: (i, 0, 0))],
        out_specs=pl.BlockSpec((_G, d), lambda i: (0, 0)),
        out_shape=jax.ShapeDtypeStruct((_G, d), _F32),
        interpret=interpret,
    )(s0, s1, p0, p1, dv, b, batch_blocks)


def _mlp_call(g, wh1, bh1, wh2, bh2, interpret=False):
    out_dim = wh2.shape[1]

    def body(g_ref, w1_ref, b1_ref, w2_ref, b2_ref, o_ref):
        h = jnp.maximum(
            jnp.dot(g_ref[...], w1_ref[...], preferred_element_type=_F32)
            + b1_ref[...], 0.0)
        o_ref[...] = (
            jnp.dot(h, w2_ref[...], preferred_element_type=_F32) + b2_ref[...])

    return pl.pallas_call(
        body,
        out_shape=jax.ShapeDtypeStruct((g.shape[0], out_dim), _F32),
        interpret=interpret,
    )(g, wh1, bh1, wh2, bh2)


def kernel(x, edge_index, batch, W1, b1, W2, b2, W3, b3, Wh1, bh1, Wh2, bh2):
    n, d = x.shape
    e = edge_index.shape[1]
    r = 400

    unit = 512 * _CHUNK
    e_pad = -(-e // unit) * unit
    src_p = jnp.concatenate(
        [edge_index[0], jnp.zeros((e_pad - e,), jnp.int32)]
    ).reshape(e_pad // _CHUNK, _CHUNK)
    dst_p = jnp.concatenate(
        [edge_index[1], jnp.full((e_pad - e,), n, jnp.int32)]
    ).reshape(e_pad // _CHUNK, _CHUNK)

    d0, d1 = _degree_call(dst_p, n)
    p0, p1, dv = _stage0_call(x, W1, d0, d1, r)
    s0, s1 = _aggregate_call(src_p, dst_p, p0, p1)
    q0, q1 = _stage_mid_call(s0, s1, p0, p1, dv, b1.reshape(1, -1), W2, r)
    s0, s1 = _aggregate_call(src_p, dst_p, q0, q1)
    q0b, q1b = _stage_mid_call(s0, s1, q0, q1, dv, b2.reshape(1, -1), W3, r)
    s0, s1 = _aggregate_call(src_p, dst_p, q0b, q1b)
    g = _stage_final_call(s0, s1, q0b, q1b, dv, b3.reshape(1, -1),
                          batch.reshape(n // r, 1, r), r)
    return _mlp_call(g, Wh1, bh1.reshape(1, -1), Wh2, bh2.reshape(1, -1))


# speedup vs baseline: 1.0659x; 1.0659x over previous
import functools

import jax
import jax.numpy as jnp
from jax import lax
from jax.experimental import pallas as pl
from jax.experimental.pallas import tpu as pltpu
from jax.experimental.pallas import tpu_sc as plsc

_F32 = jnp.float32
_CHUNK = 128
_NBUF = 2
_DEG_W = 16
_G = 64


def _row_split(n_rows, n_tiles):
    base = -(-n_rows // n_tiles)
    base = -(-base // 8) * 8
    out = []
    start = 0
    for t in range(n_tiles):
        sz = min(base, n_rows - start)
        if sz <= 0:
            break
        out.append((start, sz))
        start += sz
    return out


def _copy_chunks(src_ref, dst_ref, dst_start, total, buf_rows):
    off = 0
    while off < total:
        sz = min(buf_rows, total - off)
        pltpu.sync_copy(src_ref.at[pl.ds(0, sz)], dst_ref.at[pl.ds(dst_start + off, sz)])
        off += sz


def _degree_call(dst_p, n, interpret=False):
    rows_total = dst_p.shape[0]
    per_core = rows_total // 2
    per_tile = per_core // 16
    acc_rows = n + 8
    out_split = _row_split(n, 16)
    zero_split = _row_split(acc_rows, 16)
    mesh = plsc.VectorSubcoreMesh(
        core_axis_name="c", subcore_axis_name="s", num_cores=2, num_subcores=16)

    @functools.partial(
        pl.kernel,
        out_type=[jax.ShapeDtypeStruct((n, _DEG_W), _F32)] * 2,
        mesh=mesh,
        interpret=interpret,
        scratch_types=[
            pltpu.VMEM((per_tile, _CHUNK), jnp.int32),
            pltpu.VMEM((_CHUNK, _DEG_W), _F32),
            pltpu.VMEM((_CHUNK, _DEG_W), _F32),
            pltpu.VMEM_SHARED((acc_rows, _DEG_W), _F32),
        ],
    )
    def deg_kernel(dst_hbm, d0_hbm, d1_hbm, didx, ones_v, zb, acc):
        cid = lax.axis_index("c")
        tid = lax.axis_index("s")

        pltpu.sync_copy(
            dst_hbm.at[pl.ds(cid * per_core + tid * per_tile, per_tile)], didx)

        def init_row(i, _):
            ones_v[i, :] = jnp.ones((_DEG_W,), _F32)
            zb[i, :] = jnp.zeros((_DEG_W,), _F32)
            return ()

        lax.fori_loop(0, _CHUNK, init_row, ())

        for t, (zs, zn) in enumerate(zero_split):
            @pl.when(tid == t)
            def _(zs=zs, zn=zn):
                _copy_chunks(zb, acc, zs, zn, _CHUNK)

        plsc.subcore_barrier()

        def body(i, _):
            pltpu.sync_copy(ones_v, acc.at[didx.at[i]], add=True)
            return ()

        lax.fori_loop(0, per_tile, body, ())
        plsc.subcore_barrier()

        def copy_out(out_hbm):
            for t, (os, on) in enumerate(out_split):
                @pl.when(tid == t)
                def _(os=os, on=on):
                    pltpu.sync_copy(acc.at[pl.ds(os, on)], out_hbm.at[pl.ds(os, on)])

        @pl.when(cid == 0)
        def _():
            copy_out(d0_hbm)

        @pl.when(cid == 1)
        def _():
            copy_out(d1_hbm)

    return deg_kernel(dst_p)


def _aggregate_call(src_p, dst_p, p0, p1, interpret=False):
    n = p0.shape[0]
    hw = p0.shape[1]
    per_tile = src_p.shape[0] // 16
    acc_rows = n + 8
    out_split = _row_split(n, 16)
    zero_split = _row_split(acc_rows, 16)
    mesh = plsc.VectorSubcoreMesh(
        core_axis_name="c", subcore_axis_name="s", num_cores=2, num_subcores=16)

    piece = per_tile // 2
    zrows = 16

    @functools.partial(
        pl.kernel,
        out_type=[jax.ShapeDtypeStruct((n, hw), _F32)] * 2,
        mesh=mesh,
        interpret=interpret,
        scratch_types=[
            pltpu.VMEM((piece, _CHUNK), jnp.int32),
            pltpu.VMEM((piece, _CHUNK), jnp.int32),
            pltpu.VMEM((_CHUNK, hw), _F32),
            pltpu.VMEM((_CHUNK, hw), _F32),
            pltpu.VMEM((zrows, hw), _F32),
            pltpu.VMEM_SHARED((acc_rows, hw), _F32),
            pltpu.SemaphoreType.DMA,
            pltpu.SemaphoreType.DMA,
        ],
    )
    def agg_kernel(src_hbm, dst_hbm, p0_hbm, p1_hbm, s0_hbm, s1_hbm,
                   sidx, didx, rows0, rows1, zb, acc, sem0, sem1):
        cid = lax.axis_index("c")
        tid = lax.axis_index("s")

        def zero_row(i, _):
            for j in range(hw // 16):
                zb[i, pl.ds(j * 16, 16)] = jnp.zeros((16,), _F32)
            return ()

        lax.fori_loop(0, zrows, zero_row, ())

        for t, (zs, zn) in enumerate(zero_split):
            @pl.when(tid == t)
            def _(zs=zs, zn=zn):
                _copy_chunks(zb, acc, zs, zn, zrows)

        plsc.subcore_barrier()

        def edge_loop(table_hbm):
            bufs = (rows0, rows1)
            sems = (sem0, sem1)
            for q in range(per_tile // piece):
                base = tid * per_tile + q * piece
                pltpu.sync_copy(src_hbm.at[pl.ds(base, piece)], sidx)
                pltpu.sync_copy(dst_hbm.at[pl.ds(base, piece)], didx)
                for b in range(_NBUF):
                    pltpu.async_copy(table_hbm.at[sidx.at[b]], bufs[b], sems[b])

                def body(g, _):
                    for b in range(_NBUF):
                        cur = g * _NBUF + b
                        pltpu.make_async_copy(
                            table_hbm.at[pl.ds(0, _CHUNK)],
                            bufs[b], sems[b]).wait()
                        pltpu.sync_copy(bufs[b], acc.at[didx.at[cur]], add=True)
                        nxt = cur + _NBUF

                        @pl.when(nxt < piece)
                        def _(b=b, nxt=nxt):
                            pltpu.async_copy(
                                table_hbm.at[sidx.at[nxt]], bufs[b], sems[b])
                    return ()

                lax.fori_loop(0, piece // _NBUF, body, ())

        @pl.when(cid == 0)
        def _():
            edge_loop(p0_hbm)

        @pl.when(cid == 1)
        def _():
            edge_loop(p1_hbm)

        plsc.subcore_barrier()

        def copy_out(out_hbm):
            for t, (os, on) in enumerate(out_split):
                @pl.when(tid == t)
                def _(os=os, on=on):
                    pltpu.sync_copy(acc.at[pl.ds(os, on)], out_hbm.at[pl.ds(os, on)])

        @pl.when(cid == 0)
        def _():
            copy_out(s0_hbm)

        @pl.when(cid == 1)
        def _():
            copy_out(s1_hbm)

    return agg_kernel(src_p, dst_p, p0, p1)


def _stage0_call(x, w1, d0, d1, r, interpret=False):
    n, d = x.shape

    def body(x_ref, w_ref, d0_ref, d1_ref, p0_ref, p1_ref, dv_ref):
        dinv = lax.rsqrt(d0_ref[:, 0:1] + d1_ref[:, 0:1] + 1.0)
        y = jnp.dot(x_ref[...], w_ref[...], preferred_element_type=_F32)
        p = y * dinv
        p0_ref[...] = p[:, : d // 2]
        p1_ref[...] = p[:, d // 2:]
        dv_ref[...] = dinv

    return pl.pallas_call(
        body,
        grid=(n // r,),
        in_specs=[
            pl.BlockSpec((r, d), lambda i: (i, 0)),
            pl.BlockSpec((d, d), lambda i: (0, 0)),
            pl.BlockSpec((r, _DEG_W), lambda i: (i, 0)),
            pl.BlockSpec((r, _DEG_W), lambda i: (i, 0)),
        ],
        out_specs=[
            pl.BlockSpec((r, d // 2), lambda i: (i, 0)),
            pl.BlockSpec((r, d // 2), lambda i: (i, 0)),
            pl.BlockSpec((r, 1), lambda i: (i, 0)),
        ],
        out_shape=[
            jax.ShapeDtypeStruct((n, d // 2), _F32),
            jax.ShapeDtypeStruct((n, d // 2), _F32),
            jax.ShapeDtypeStruct((n, 1), _F32),
        ],
        interpret=interpret,
    )(x, w1, d0, d1)


def _stage_mid_call(s0, s1, p0, p1, dv, b, w, r, interpret=False):
    n, hw = s0.shape
    d = 2 * hw

    def body(s0_ref, s1_ref, p0_ref, p1_ref, dv_ref, b_ref, w_ref, q0_ref, q1_ref):
        t = jnp.concatenate(
            [s0_ref[...] + p0_ref[...], s1_ref[...] + p1_ref[...]], axis=1)
        h = jnp.maximum(dv_ref[...] * t + b_ref[...], 0.0)
        y = jnp.dot(h, w_ref[...], preferred_element_type=_F32)
        q = y * dv_ref[...]
        q0_ref[...] = q[:, :hw]
        q1_ref[...] = q[:, hw:]

    half = pl.BlockSpec((r, hw), lambda i: (i, 0))
    return pl.pallas_call(
        body,
        grid=(n // r,),
        in_specs=[half, half, half, half,
                  pl.BlockSpec((r, 1), lambda i: (i, 0)),
                  pl.BlockSpec((1, d), lambda i: (0, 0)),
                  pl.BlockSpec((d, d), lambda i: (0, 0))],
        out_specs=[half, half],
        out_shape=[jax.ShapeDtypeStruct((n, hw), _F32)] * 2,
        interpret=interpret,
    )(s0, s1, p0, p1, dv, b, w)


def _stage_final_call(s0, s1, p0, p1, dv, b, batch_blocks, r, interpret=False):
    n, hw = s0.shape
    d = 2 * hw

    def body(s0_ref, s1_ref, p0_ref, p1_ref, dv_ref, b_ref, bat_ref, g_ref):
        i = pl.program_id(0)
        t = jnp.concatenate(
            [s0_ref[...] + p0_ref[...], s1_ref[...] + p1_ref[...]], axis=1)
        h = jnp.maximum(dv_ref[...] * t + b_ref[...], 0.0)
        ids = jnp.broadcast_to(bat_ref[0], (_G, r))
        onehot = (lax.broadcasted_iota(jnp.int32, (_G, r), 0) == ids).astype(_F32)
        gp = jnp.dot(onehot, h, preferred_element_type=_F32)

        @pl.when(i == 0)
        def _():
            g_ref[...] = gp

        @pl.when(i > 0)
        def _():
            g_ref[...] += gp

    half = pl.BlockSpec((r, hw), lambda i: (i, 0))
    return pl.pallas_call(
        body,
        grid=(n // r,),
        in_specs=[half, half, half, half,
                  pl.BlockSpec((r, 1), lambda i: (i, 0)),
                  pl.BlockSpec((1, d), lambda i: (0, 0)),
                  pl.BlockSpec((1, 1, r), lambda i: (i, 0, 0))],
        out_specs=pl.BlockSpec((_G, d), lambda i: (0, 0)),
        out_shape=jax.ShapeDtypeStruct((_G, d), _F32),
        interpret=interpret,
    )(s0, s1, p0, p1, dv, b, batch_blocks)


def _mlp_call(g, wh1, bh1, wh2, bh2, interpret=False):
    out_dim = wh2.shape[1]

    def body(g_ref, w1_ref, b1_ref, w2_ref, b2_ref, o_ref):
        h = jnp.maximum(
            jnp.dot(g_ref[...], w1_ref[...], preferred_element_type=_F32)
            + b1_ref[...], 0.0)
        o_ref[...] = (
            jnp.dot(h, w2_ref[...], preferred_element_type=_F32) + b2_ref[...])

    return pl.pallas_call(
        body,
        out_shape=jax.ShapeDtypeStruct((g.shape[0], out_dim), _F32),
        interpret=interpret,
    )(g, wh1, bh1, wh2, bh2)


def kernel(x, edge_index, batch, W1, b1, W2, b2, W3, b3, Wh1, bh1, Wh2, bh2):
    n, d = x.shape
    e = edge_index.shape[1]
    r = 1000

    unit = 256 * _CHUNK
    e_pad = -(-e // unit) * unit
    src_p = jnp.concatenate(
        [edge_index[0], jnp.zeros((e_pad - e,), jnp.int32)]
    ).reshape(e_pad // _CHUNK, _CHUNK)
    dst_p = jnp.concatenate(
        [edge_index[1], jnp.full((e_pad - e,), n, jnp.int32)]
    ).reshape(e_pad // _CHUNK, _CHUNK)

    d0, d1 = _degree_call(dst_p, n)
    p0, p1, dv = _stage0_call(x, W1, d0, d1, r)
    s0, s1 = _aggregate_call(src_p, dst_p, p0, p1)
    q0, q1 = _stage_mid_call(s0, s1, p0, p1, dv, b1.reshape(1, -1), W2, r)
    s0, s1 = _aggregate_call(src_p, dst_p, q0, q1)
    q0b, q1b = _stage_mid_call(s0, s1, q0, q1, dv, b2.reshape(1, -1), W3, r)
    s0, s1 = _aggregate_call(src_p, dst_p, q0b, q1b)
    g = _stage_final_call(s0, s1, q0b, q1b, dv, b3.reshape(1, -1),
                          batch.reshape(n // r, 1, r), r)
    return _mlp_call(g, Wh1, bh1.reshape(1, -1), Wh2, bh2.reshape(1, -1))


# speedup vs baseline: 1.0852x; 1.0181x over previous
import functools

import jax
import jax.numpy as jnp
from jax import lax
from jax.experimental import pallas as pl
from jax.experimental.pallas import tpu as pltpu
from jax.experimental.pallas import tpu_sc as plsc

_F32 = jnp.float32
_CHUNK = 128
_NBUF = 2
_DEG_W = 16
_G = 64


def _row_split(n_rows, n_tiles):
    base = -(-n_rows // n_tiles)
    base = -(-base // 8) * 8
    out = []
    start = 0
    for t in range(n_tiles):
        sz = min(base, n_rows - start)
        if sz <= 0:
            break
        out.append((start, sz))
        start += sz
    return out


def _copy_chunks(src_ref, dst_ref, dst_start, total, buf_rows):
    off = 0
    while off < total:
        sz = min(buf_rows, total - off)
        pltpu.sync_copy(src_ref.at[pl.ds(0, sz)], dst_ref.at[pl.ds(dst_start + off, sz)])
        off += sz


def _degree_call(dst_p, n, interpret=False):
    rows_total = dst_p.shape[0]
    per_core = rows_total // 2
    per_tile = per_core // 16
    acc_rows = n + 8
    out_split = _row_split(n, 16)
    zero_split = _row_split(acc_rows, 16)
    mesh = plsc.VectorSubcoreMesh(
        core_axis_name="c", subcore_axis_name="s", num_cores=2, num_subcores=16)

    @functools.partial(
        pl.kernel,
        out_type=[jax.ShapeDtypeStruct((n, _DEG_W), _F32)] * 2,
        mesh=mesh,
        interpret=interpret,
        scratch_types=[
            pltpu.VMEM((per_tile, _CHUNK), jnp.int32),
            pltpu.VMEM((_CHUNK, _DEG_W), _F32),
            pltpu.VMEM((_CHUNK, _DEG_W), _F32),
            pltpu.VMEM_SHARED((acc_rows, _DEG_W), _F32),
        ],
    )
    def deg_kernel(dst_hbm, d0_hbm, d1_hbm, didx, ones_v, zb, acc):
        cid = lax.axis_index("c")
        tid = lax.axis_index("s")

        pltpu.sync_copy(
            dst_hbm.at[pl.ds(cid * per_core + tid * per_tile, per_tile)], didx)

        def init_row(i, _):
            ones_v[i, :] = jnp.ones((_DEG_W,), _F32)
            zb[i, :] = jnp.zeros((_DEG_W,), _F32)
            return ()

        lax.fori_loop(0, _CHUNK, init_row, ())

        for t, (zs, zn) in enumerate(zero_split):
            @pl.when(tid == t)
            def _(zs=zs, zn=zn):
                _copy_chunks(zb, acc, zs, zn, _CHUNK)

        plsc.subcore_barrier()

        def body(i, _):
            pltpu.sync_copy(ones_v, acc.at[didx.at[i]], add=True)
            return ()

        lax.fori_loop(0, per_tile, body, ())
        plsc.subcore_barrier()

        def copy_out(out_hbm):
            for t, (os, on) in enumerate(out_split):
                @pl.when(tid == t)
                def _(os=os, on=on):
                    pltpu.sync_copy(acc.at[pl.ds(os, on)], out_hbm.at[pl.ds(os, on)])

        @pl.when(cid == 0)
        def _():
            copy_out(d0_hbm)

        @pl.when(cid == 1)
        def _():
            copy_out(d1_hbm)

    return deg_kernel(dst_p)


def _aggregate_call(src_p, dst_p, p0, p1, interpret=False):
    n = p0.shape[0]
    hw = p0.shape[1]
    per_tile = src_p.shape[0] // 16
    acc_rows = n + 8
    out_split = _row_split(n, 16)
    zero_split = _row_split(acc_rows, 16)
    mesh = plsc.VectorSubcoreMesh(
        core_axis_name="c", subcore_axis_name="s", num_cores=2, num_subcores=16)

    piece = per_tile // 2
    zrows = 48

    @functools.partial(
        pl.kernel,
        out_type=[jax.ShapeDtypeStruct((n, hw), _F32)] * 2,
        mesh=mesh,
        interpret=interpret,
        scratch_types=[
            pltpu.VMEM((piece, _CHUNK), jnp.int32),
            pltpu.VMEM((piece, _CHUNK), jnp.int32),
            pltpu.VMEM((_CHUNK, hw), _F32),
            pltpu.VMEM((_CHUNK, hw), _F32),
            pltpu.VMEM((zrows, hw), _F32),
            pltpu.VMEM_SHARED((acc_rows, hw), _F32),
            pltpu.SemaphoreType.DMA,
            pltpu.SemaphoreType.DMA,
        ],
    )
    def agg_kernel(src_hbm, dst_hbm, p0_hbm, p1_hbm, s0_hbm, s1_hbm,
                   sidx, didx, rows0, rows1, zb, acc, sem0, sem1):
        cid = lax.axis_index("c")
        tid = lax.axis_index("s")

        def zero_row(i, _):
            for j in range(hw // 16):
                zb[i, pl.ds(j * 16, 16)] = jnp.zeros((16,), _F32)
            return ()

        lax.fori_loop(0, zrows, zero_row, ())

        for t, (zs, zn) in enumerate(zero_split):
            @pl.when(tid == t)
            def _(zs=zs, zn=zn):
                _copy_chunks(zb, acc, zs, zn, zrows)

        plsc.subcore_barrier()

        def edge_loop(table_hbm):
            bufs = (rows0, rows1)
            sems = (sem0, sem1)
            for q in range(per_tile // piece):
                base = tid * per_tile + q * piece
                pltpu.sync_copy(src_hbm.at[pl.ds(base, piece)], sidx)
                pltpu.sync_copy(dst_hbm.at[pl.ds(base, piece)], didx)
                for b in range(_NBUF):
                    pltpu.async_copy(table_hbm.at[sidx.at[b]], bufs[b], sems[b])

                def body(g, _):
                    for b in range(_NBUF):
                        cur = g * _NBUF + b
                        pltpu.make_async_copy(
                            table_hbm.at[pl.ds(0, _CHUNK)],
                            bufs[b], sems[b]).wait()
                        pltpu.sync_copy(bufs[b], acc.at[didx.at[cur]], add=True)
                        nxt = cur + _NBUF

                        @pl.when(nxt < piece)
                        def _(b=b, nxt=nxt):
                            pltpu.async_copy(
                                table_hbm.at[sidx.at[nxt]], bufs[b], sems[b])
                    return ()

                lax.fori_loop(0, piece // _NBUF, body, ())

        @pl.when(cid == 0)
        def _():
            edge_loop(p0_hbm)

        @pl.when(cid == 1)
        def _():
            edge_loop(p1_hbm)

        plsc.subcore_barrier()

        def copy_out(out_hbm):
            for t, (os, on) in enumerate(out_split):
                @pl.when(tid == t)
                def _(os=os, on=on):
                    pltpu.sync_copy(acc.at[pl.ds(os, on)], out_hbm.at[pl.ds(os, on)])

        @pl.when(cid == 0)
        def _():
            copy_out(s0_hbm)

        @pl.when(cid == 1)
        def _():
            copy_out(s1_hbm)

    return agg_kernel(src_p, dst_p, p0, p1)


def _stage0_call(x, w1, d0, d1, r, interpret=False):
    n, d = x.shape

    def body(x_ref, w_ref, d0_ref, d1_ref, p0_ref, p1_ref, dv_ref):
        dinv = lax.rsqrt(d0_ref[:, 0:1] + d1_ref[:, 0:1] + 1.0)
        y = jnp.dot(x_ref[...], w_ref[...], preferred_element_type=_F32)
        p = y * dinv
        p0_ref[...] = p[:, : d // 2]
        p1_ref[...] = p[:, d // 2:]
        dv_ref[...] = dinv

    return pl.pallas_call(
        body,
        grid=(n // r,),
        in_specs=[
            pl.BlockSpec((r, d), lambda i: (i, 0)),
            pl.BlockSpec((d, d), lambda i: (0, 0)),
            pl.BlockSpec((r, _DEG_W), lambda i: (i, 0)),
            pl.BlockSpec((r, _DEG_W), lambda i: (i, 0)),
        ],
        out_specs=[
            pl.BlockSpec((r, d // 2), lambda i: (i, 0)),
            pl.BlockSpec((r, d // 2), lambda i: (i, 0)),
            pl.BlockSpec((r, 1), lambda i: (i, 0)),
        ],
        out_shape=[
            jax.ShapeDtypeStruct((n, d // 2), _F32),
            jax.ShapeDtypeStruct((n, d // 2), _F32),
            jax.ShapeDtypeStruct((n, 1), _F32),
        ],
        interpret=interpret,
    )(x, w1, d0, d1)


def _stage_mid_call(s0, s1, p0, p1, dv, b, w, r, interpret=False):
    n, hw = s0.shape
    d = 2 * hw

    def body(s0_ref, s1_ref, p0_ref, p1_ref, dv_ref, b_ref, w_ref, q0_ref, q1_ref):
        t = jnp.concatenate(
            [s0_ref[...] + p0_ref[...], s1_ref[...] + p1_ref[...]], axis=1)
        h = jnp.maximum(dv_ref[...] * t + b_ref[...], 0.0)
        y = jnp.dot(h, w_ref[...], preferred_element_type=_F32)
        q = y * dv_ref[...]
        q0_ref[...] = q[:, :hw]
        q1_ref[...] = q[:, hw:]

    half = pl.BlockSpec((r, hw), lambda i: (i, 0))
    return pl.pallas_call(
        body,
        grid=(n // r,),
        in_specs=[half, half, half, half,
                  pl.BlockSpec((r, 1), lambda i: (i, 0)),
                  pl.BlockSpec((1, d), lambda i: (0, 0)),
                  pl.BlockSpec((d, d), lambda i: (0, 0))],
        out_specs=[half, half],
        out_shape=[jax.ShapeDtypeStruct((n, hw), _F32)] * 2,
        interpret=interpret,
    )(s0, s1, p0, p1, dv, b, w)


def _stage_final_call(s0, s1, p0, p1, dv, b, batch_blocks,
                      wh1, bh1, wh2, bh2, r, interpret=False):
    n, hw = s0.shape
    d = 2 * hw
    nsteps = n // r
    out_dim = wh2.shape[1]

    def body(s0_ref, s1_ref, p0_ref, p1_ref, dv_ref, b_ref, bat_ref,
             w1_ref, b1_ref, w2_ref, b2_ref, o_ref, g_ref):
        i = pl.program_id(0)
        t = jnp.concatenate(
            [s0_ref[...] + p0_ref[...], s1_ref[...] + p1_ref[...]], axis=1)
        h = jnp.maximum(dv_ref[...] * t + b_ref[...], 0.0)
        ids = jnp.broadcast_to(bat_ref[0], (_G, r))
        onehot = (lax.broadcasted_iota(jnp.int32, (_G, r), 0) == ids).astype(_F32)
        gp = jnp.dot(onehot, h, preferred_element_type=_F32)

        @pl.when(i == 0)
        def _():
            g_ref[...] = gp

        @pl.when(i > 0)
        def _():
            g_ref[...] += gp

        @pl.when(i == nsteps - 1)
        def _():
            hm = jnp.maximum(
                jnp.dot(g_ref[...], w1_ref[...], preferred_element_type=_F32)
                + b1_ref[...], 0.0)
            o_ref[...] = (
                jnp.dot(hm, w2_ref[...], preferred_element_type=_F32)
                + b2_ref[...])

    half = pl.BlockSpec((r, hw), lambda i: (i, 0))
    full = lambda shape: pl.BlockSpec(shape, lambda i: tuple(0 for _ in shape))
    return pl.pallas_call(
        body,
        grid=(nsteps,),
        in_specs=[half, half, half, half,
                  pl.BlockSpec((r, 1), lambda i: (i, 0)),
                  full((1, d)),
                  pl.BlockSpec((1, 1, r), lambda i: (i, 0, 0)),
                  full(wh1.shape), full((1, wh1.shape[1])),
                  full(wh2.shape), full((1, out_dim))],
        out_specs=pl.BlockSpec((_G, out_dim), lambda i: (0, 0)),
        out_shape=jax.ShapeDtypeStruct((_G, out_dim), _F32),
        scratch_shapes=[pltpu.VMEM((_G, d), _F32)],
        interpret=interpret,
    )(s0, s1, p0, p1, dv, b, batch_blocks, wh1, bh1, wh2, bh2)


def kernel(x, edge_index, batch, W1, b1, W2, b2, W3, b3, Wh1, bh1, Wh2, bh2):
    n, d = x.shape
    e = edge_index.shape[1]
    r = 1000

    unit = 256 * _CHUNK
    e_pad = -(-e // unit) * unit
    src_p = jnp.concatenate(
        [edge_index[0], jnp.zeros((e_pad - e,), jnp.int32)]
    ).reshape(e_pad // _CHUNK, _CHUNK)
    dst_p = jnp.concatenate(
        [edge_index[1], jnp.full((e_pad - e,), n, jnp.int32)]
    ).reshape(e_pad // _CHUNK, _CHUNK)

    d0, d1 = _degree_call(dst_p, n)
    p0, p1, dv = _stage0_call(x, W1, d0, d1, r)
    s0, s1 = _aggregate_call(src_p, dst_p, p0, p1)
    q0, q1 = _stage_mid_call(s0, s1, p0, p1, dv, b1.reshape(1, -1), W2, r)
    s0, s1 = _aggregate_call(src_p, dst_p, q0, q1)
    q0b, q1b = _stage_mid_call(s0, s1, q0, q1, dv, b2.reshape(1, -1), W3, r)
    s0, s1 = _aggregate_call(src_p, dst_p, q0b, q1b)
    return _stage_final_call(s0, s1, q0b, q1b, dv, b3.reshape(1, -1),
                             batch.reshape(n // r, 1, r),
                             Wh1, bh1.reshape(1, -1), Wh2, bh2.reshape(1, -1), r)
